# baseline (speedup 1.0000x reference)
import jax
import jax.numpy as jnp
from jax import lax
from jax.experimental import pallas as pl
from jax.experimental.pallas import tpu as pltpu
from jax.experimental.pallas import tpu_sc as plsc

C = 128
E = 8
H = 8
D = 16
T = 32
NEG = -1e9
EPS = 1e-5
CAP = 4096
NSLOT = E * CAP
NW = 32
CH = 128


def _attn_body(x_ref, g_ref, b_ref, wq_ref, wk_ref, wv_ref, wp_ref, bp_ref, o_ref):
    xb = x_ref[...]
    BB = xb.shape[0]
    G = 8
    GT = G * T
    NG = BB // G
    mu = jnp.mean(xb, axis=-1, keepdims=True)
    var = jnp.mean((xb - mu) ** 2, axis=-1, keepdims=True)
    xn = (xb - mu) / jnp.sqrt(var + EPS) * g_ref[...] + b_ref[...]
    x2 = xn.reshape(BB * T, C)
    q = jnp.dot(x2, wq_ref[...], preferred_element_type=jnp.float32).reshape(NG, GT, C)
    k = jnp.dot(x2, wk_ref[...], preferred_element_type=jnp.float32).reshape(NG, GT, C)
    v = jnp.dot(x2, wv_ref[...], preferred_element_type=jnp.float32).reshape(NG, GT, C)
    rr = jax.lax.broadcasted_iota(jnp.int32, (GT, GT), 0)
    cc = jax.lax.broadcasted_iota(jnp.int32, (GT, GT), 1)
    mask = ((rr >> 5) == (cc >> 5)) & ((rr & 31) >= (cc & 31))
    scale = C ** -0.5
    for g in range(NG):
        ohs = []
        for h in range(H):
            qh = q[g, :, h * D:(h + 1) * D]
            kh = k[g, :, h * D:(h + 1) * D]
            vh = v[g, :, h * D:(h + 1) * D]
            s = jax.lax.dot_general(qh, kh, (((1,), (1,)), ((), ())),
                                    preferred_element_type=jnp.float32) * scale
            s = jnp.where(mask, s, NEG)
            m = jnp.max(s, axis=-1, keepdims=True)
            e = jnp.exp(s - m)
            p = e / jnp.sum(e, axis=-1, keepdims=True)
            ohs.append(jnp.dot(p, vh, preferred_element_type=jnp.float32))
        og = jnp.concatenate(ohs, axis=-1)
        y = jnp.dot(og, wp_ref[...], preferred_element_type=jnp.float32) + bp_ref[...]
        o_ref[g * G:(g + 1) * G] = (xb[g * G:(g + 1) * G]
                                    + y.reshape(G, T, C))


def _router_body(x1_ref, g_ref, b_ref, wcat_ref, bcat_ref, nz_ref, tri_ref,
                 h_ref, pos_ref, mask_ref, gate_ref, aux_ref, cnt_ref, acc_ref):
    i = pl.program_id(0)

    @pl.when(i == 0)
    def _():
        acc_ref[...] = jnp.zeros_like(acc_ref)

    xb = x1_ref[...]
    mu = jnp.mean(xb, axis=-1, keepdims=True)
    var = jnp.mean((xb - mu) ** 2, axis=-1, keepdims=True)
    hh = (xb - mu) / jnp.sqrt(var + EPS) * g_ref[...] + b_ref[...]
    h_ref[...] = hh

    lc = jnp.dot(hh, wcat_ref[...], preferred_element_type=jnp.float32) + bcat_ref[...]
    logits = lc[:, 0:E]
    nlog = lc[:, E:2 * E]
    sk = lc[:, 2 * E:2 * E + 1]
    softp = jnp.logaddexp(nlog, 0.0)
    noisy = logits + nz_ref[...] * softp

    r8 = jax.lax.broadcasted_iota(jnp.int32, (E, E), 0)
    c8 = jax.lax.broadcasted_iota(jnp.int32, (E, E), 1)
    l8 = (r8 <= c8).astype(jnp.float32)
    m1 = jnp.max(noisy, axis=-1, keepdims=True)
    eq1 = (noisy == m1).astype(jnp.float32)
    cs1 = jnp.dot(eq1, l8, preferred_element_type=jnp.float32)
    first = (eq1 > 0) & (cs1 == 1.0)
    noisy2 = jnp.where(first, -3e38, noisy)
    m2 = jnp.max(noisy2, axis=-1, keepdims=True)
    eq2 = (noisy2 == m2).astype(jnp.float32)
    cs2 = jnp.dot(eq2, l8, preferred_element_type=jnp.float32)
    second = (eq2 > 0) & (cs2 == 1.0)
    topk = first | second

    z = jnp.where(topk, jnp.exp(noisy - m1), 0.0)
    gate = z / jnp.sum(z, axis=-1, keepdims=True)
    gate_ref[...] = gate

    ns = (jax.nn.sigmoid(sk) <= 0.5).astype(jnp.float32)
    emask = topk.astype(jnp.float32) * ns
    mask_ref[...] = emask

    posb = jnp.dot(tri_ref[...], emask, preferred_element_type=jnp.float32)
    pos = posb + acc_ref[0:1, 0:E]
    pos_ref[...] = pos

    lane = jax.lax.broadcasted_iota(jnp.int32, emask.shape, 1)
    aux_ref[...] = jnp.where(lane == 0, ns, 0.0)

    new_cnt = acc_ref[0:1, 0:E] + jnp.sum(emask, axis=0, keepdims=True)
    new_ns = acc_ref[0:1, E:E + 1] + jnp.sum(ns, keepdims=True).reshape(1, 1)
    rowc = jnp.concatenate([new_cnt, new_ns, jnp.zeros((1, 16 - E - 1), jnp.float32)], axis=1)
    acc_ref[...] = rowc
    cnt_ref[...] = rowc


def _slots_body(cnt_ref, pos_ref, mask_ref, gate_ref,
                i0_ref, i1_ref, w0_ref, w1_ref):
    i = pl.program_id(0)
    tglob = (i * 512.0
             + jax.lax.broadcasted_iota(jnp.int32, (512, 1), 0).astype(jnp.float32))
    sent = float(NSLOT) + tglob
    ntok = cnt_ref[0, E]
    cap = jnp.floor(ntok * 2.0 / 8.0)
    maskf = mask_ref[...]
    pos = pos_ref[...]
    gate = gate_ref[...]
    r8 = jax.lax.broadcasted_iota(jnp.int32, (E, E), 0)
    c8 = jax.lax.broadcasted_iota(jnp.int32, (E, E), 1)
    l8 = (r8 <= c8).astype(jnp.float32)
    cs = jnp.dot(maskf, l8, preferred_element_type=jnp.float32)
    firstb = (maskf > 0) & (cs == 1.0)
    secondb = (maskf > 0) & (cs == 2.0)
    within = pos <= cap
    lanef = jax.lax.broadcasted_iota(jnp.int32, maskf.shape, 1).astype(jnp.float32)
    slotv = lanef * CAP + pos - 1.0

    sel0 = firstb & within
    sel0f = sel0.astype(jnp.float32)
    has0 = jnp.sum(sel0f, axis=-1, keepdims=True)
    d0 = jnp.sum(sel0f * slotv, axis=-1, keepdims=True)
    d0 = jnp.where(has0 > 0, d0, sent)
    w0 = jnp.sum(sel0f * gate, axis=-1, keepdims=True)

    sel1 = secondb & within
    sel1f = sel1.astype(jnp.float32)
    has1 = jnp.sum(sel1f, axis=-1, keepdims=True)
    d1 = jnp.sum(sel1f * slotv, axis=-1, keepdims=True)
    d1 = jnp.where(has1 > 0, d1, sent)
    w1 = jnp.sum(sel1f * gate, axis=-1, keepdims=True)

    i0_ref[...] = d0.astype(jnp.int32)
    i1_ref[...] = d1.astype(jnp.int32)
    w0_ref[...] = w0
    w1_ref[...] = w1


def _ffn_body(cnt_ref, buf_ref, w1_ref, b1_ref, w2_ref, b2_ref, o_ref):
    e = pl.program_id(0)
    j = pl.program_id(1)
    lane = jax.lax.broadcasted_iota(jnp.int32, (1, 16), 1)
    cnte = jnp.sum(jnp.where(lane == e, cnt_ref[...], 0.0))
    ntok = cnt_ref[0, E]
    cap = jnp.floor(ntok * 2.0 / 8.0)
    used = jnp.minimum(cnte, cap)

    @pl.when((j * 512.0) < used)
    def _():
        hh = buf_ref[...].astype(jnp.bfloat16)
        t = jnp.maximum(jnp.dot(hh, w1_ref[0], preferred_element_type=jnp.float32)
                        + b1_ref[0], 0.0)
        o_ref[...] = jnp.dot(t.astype(jnp.bfloat16), w2_ref[0],
                             preferred_element_type=jnp.float32) + b2_ref[0]


def _combine_body(x1_ref, h_ref, g0_ref, g1_ref, w0_ref, w1_ref, aux_ref, o_ref):
    w0 = w0_ref[...]
    w1 = w1_ref[...]
    upd = (jnp.where(w0 > 0, g0_ref[...], 0.0) * w0
           + jnp.where(w1 > 0, g1_ref[...], 0.0) * w1)
    ns = aux_ref[:, 0:1]
    o_ref[...] = x1_ref[...] + jnp.where(ns > 0.5, upd, h_ref[...])


def _make_sc_scatter():
    mesh = plsc.VectorSubcoreMesh(core_axis_name="c", subcore_axis_name="s")

    @pl.kernel(
        mesh=mesh,
        out_type=jax.ShapeDtypeStruct((NSLOT + 16384 + 8, C), jnp.float32),
        scratch_types=[
            pltpu.VMEM((512, C), jnp.float32),
            pltpu.VMEM((4, CH), jnp.int32),
            pltpu.VMEM((4, CH), jnp.int32),
            pltpu.SemaphoreType.DMA,
            pltpu.SemaphoreType.DMA,
        ],
    )
    def sc_scatter(h_hbm, i0_hbm, i1_hbm, buf_hbm, rows_v, idx0_v, idx1_v, semi, sem):
        wid = lax.axis_index("s") * 2 + lax.axis_index("c")
        base = wid * 512
        c0 = pltpu.async_copy(h_hbm.at[pl.ds(base, 512)], rows_v, semi)
        c1 = pltpu.async_copy(i0_hbm.at[wid], idx0_v, semi)
        c2 = pltpu.async_copy(i1_hbm.at[wid], idx1_v, semi)
        c0.wait()
        c1.wait()
        c2.wait()
        cps = []
        for j in range(4):
            src = rows_v.at[pl.ds(j * CH, CH)]
            cps.append(pltpu.async_copy(src, buf_hbm.at[idx0_v.at[j]], sem))
            cps.append(pltpu.async_copy(src, buf_hbm.at[idx1_v.at[j]], sem))
        for cp in cps:
            cp.wait()

    return sc_scatter


def _make_sc_gather():
    mesh = plsc.VectorSubcoreMesh(core_axis_name="c", subcore_axis_name="s")

    @pl.kernel(
        mesh=mesh,
        out_type=[
            jax.ShapeDtypeStruct((NW * 512, C), jnp.float32),
            jax.ShapeDtypeStruct((NW * 512, C), jnp.float32),
        ],
        scratch_types=[
            pltpu.VMEM((256, C), jnp.float32),
            pltpu.VMEM((256, C), jnp.float32),
            pltpu.VMEM((4, CH), jnp.int32),
            pltpu.VMEM((4, CH), jnp.int32),
            pltpu.SemaphoreType.DMA,
            pltpu.SemaphoreType.DMA,
        ],
    )
    def sc_gather(fbuf_hbm, i0_hbm, i1_hbm, g0_hbm, g1_hbm,
                  rows0_v, rows1_v, idx0_v, idx1_v, semi, sem):
        wid = lax.axis_index("s") * 2 + lax.axis_index("c")
        base = wid * 512
        c1 = pltpu.async_copy(i0_hbm.at[wid], idx0_v, semi)
        c2 = pltpu.async_copy(i1_hbm.at[wid], idx1_v, semi)
        c1.wait()
        c2.wait()
        for half in range(2):
            cps = []
            for j in range(2):
                jj = half * 2 + j
                cps.append(pltpu.async_copy(
                    fbuf_hbm.at[idx0_v.at[jj]], rows0_v.at[pl.ds(j * CH, CH)], sem))
                cps.append(pltpu.async_copy(
                    fbuf_hbm.at[idx1_v.at[jj]], rows1_v.at[pl.ds(j * CH, CH)], sem))
            for cp in cps:
                cp.wait()
            o0 = pltpu.async_copy(
                rows0_v, g0_hbm.at[pl.ds(base + half * 256, 256)], semi)
            o1 = pltpu.async_copy(
                rows1_v, g1_hbm.at[pl.ds(base + half * 256, 256)], semi)
            o0.wait()
            o1.wait()

    return sc_gather


def kernel(x, ln1_g, ln1_b, Wq, Wk, Wv, Wp, bp, ln2_g, ln2_b, We, be, Wn, bn,
           Ws, bs, eW1, eb1, eW2, eb2):
    B = x.shape[0]
    N = B * T
    BB = 64
    BT = 512
    nb = B // BB
    nt = N // BT

    wq = Wq.transpose(1, 0, 2).reshape(C, C)
    wk = Wk.transpose(1, 0, 2).reshape(C, C)
    wv = Wv.transpose(1, 0, 2).reshape(C, C)
    g1 = ln1_g.reshape(1, C)
    b1 = ln1_b.reshape(1, C)
    bpr = bp.reshape(1, C)

    x1 = pl.pallas_call(
        _attn_body,
        grid=(nb,),
        in_specs=[
            pl.BlockSpec((BB, T, C), lambda i: (i, 0, 0)),
            pl.BlockSpec((1, C), lambda i: (0, 0)),
            pl.BlockSpec((1, C), lambda i: (0, 0)),
            pl.BlockSpec((C, C), lambda i: (0, 0)),
            pl.BlockSpec((C, C), lambda i: (0, 0)),
            pl.BlockSpec((C, C), lambda i: (0, 0)),
            pl.BlockSpec((C, C), lambda i: (0, 0)),
            pl.BlockSpec((1, C), lambda i: (0, 0)),
        ],
        out_specs=pl.BlockSpec((BB, T, C), lambda i: (i, 0, 0)),
        out_shape=jax.ShapeDtypeStruct((B, T, C), jnp.float32),
    )(x, g1, b1, wq, wk, wv, Wp, bpr)

    x1f = x1.reshape(N, C)
    wcat = jnp.zeros((C, 32), jnp.float32)
    wcat = wcat.at[:, 0:E].set(We).at[:, E:2 * E].set(Wn).at[:, 2 * E:2 * E + 1].set(Ws)
    bcat = jnp.zeros((1, 32), jnp.float32)
    bcat = bcat.at[0, 0:E].set(be).at[0, E:2 * E].set(bn).at[0, 2 * E:2 * E + 1].set(bs)
    nz = jax.random.normal(jax.random.key(42), (B, T, E), dtype=jnp.float32).reshape(N, E)
    tri = jnp.tril(jnp.ones((BT, BT), jnp.float32))
    g2 = ln2_g.reshape(1, C)
    b2 = ln2_b.reshape(1, C)

    h, pos, maskf, gate, aux, cnt = pl.pallas_call(
        _router_body,
        grid=(nt,),
        in_specs=[
            pl.BlockSpec((BT, C), lambda i: (i, 0)),
            pl.BlockSpec((1, C), lambda i: (0, 0)),
            pl.BlockSpec((1, C), lambda i: (0, 0)),
            pl.BlockSpec((C, 32), lambda i: (0, 0)),
            pl.BlockSpec((1, 32), lambda i: (0, 0)),
            pl.BlockSpec((BT, E), lambda i: (i, 0)),
            pl.BlockSpec((BT, BT), lambda i: (0, 0)),
        ],
        out_specs=[
            pl.BlockSpec((BT, C), lambda i: (i, 0)),
            pl.BlockSpec((BT, E), lambda i: (i, 0)),
            pl.BlockSpec((BT, E), lambda i: (i, 0)),
            pl.BlockSpec((BT, E), lambda i: (i, 0)),
            pl.BlockSpec((BT, E), lambda i: (i, 0)),
            pl.BlockSpec((1, 16), lambda i: (0, 0)),
        ],
        out_shape=[
            jax.ShapeDtypeStruct((N, C), jnp.float32),
            jax.ShapeDtypeStruct((N, E), jnp.float32),
            jax.ShapeDtypeStruct((N, E), jnp.float32),
            jax.ShapeDtypeStruct((N, E), jnp.float32),
            jax.ShapeDtypeStruct((N, E), jnp.float32),
            jax.ShapeDtypeStruct((1, 16), jnp.float32),
        ],
        scratch_shapes=[pltpu.VMEM((1, 16), jnp.float32)],
    )(x1f, g2, b2, wcat, bcat, nz, tri)

    i0, i1, w0, w1 = pl.pallas_call(
        _slots_body,
        grid=(nt,),
        in_specs=[
            pl.BlockSpec((1, 16), lambda i: (0, 0)),
            pl.BlockSpec((BT, E), lambda i: (i, 0)),
            pl.BlockSpec((BT, E), lambda i: (i, 0)),
            pl.BlockSpec((BT, E), lambda i: (i, 0)),
        ],
        out_specs=[
            pl.BlockSpec((BT, 1), lambda i: (i, 0)),
            pl.BlockSpec((BT, 1), lambda i: (i, 0)),
            pl.BlockSpec((BT, 1), lambda i: (i, 0)),
            pl.BlockSpec((BT, 1), lambda i: (i, 0)),
        ],
        out_shape=[
            jax.ShapeDtypeStruct((N, 1), jnp.int32),
            jax.ShapeDtypeStruct((N, 1), jnp.int32),
            jax.ShapeDtypeStruct((N, 1), jnp.float32),
            jax.ShapeDtypeStruct((N, 1), jnp.float32),
        ],
    )(cnt, pos, maskf, gate)

    i0r = i0.reshape(NW, 4, CH)
    i1r = i1.reshape(NW, 4, CH)

    buf = _make_sc_scatter()(h, i0r, i1r)

    eb1r = eb1.reshape(E, 1, 4 * C)
    eb2r = eb2.reshape(E, 1, C)
    fbuf = pl.pallas_call(
        _ffn_body,
        grid=(E, CAP // 512),
        in_specs=[
            pl.BlockSpec((1, 16), lambda e, j: (0, 0)),
            pl.BlockSpec((512, C), lambda e, j: (e * (CAP // 512) + j, 0)),
            pl.BlockSpec((1, C, 4 * C), lambda e, j: (e, 0, 0)),
            pl.BlockSpec((1, 1, 4 * C), lambda e, j: (e, 0, 0)),
            pl.BlockSpec((1, 4 * C, C), lambda e, j: (e, 0, 0)),
            pl.BlockSpec((1, 1, C), lambda e, j: (e, 0, 0)),
        ],
        out_specs=pl.BlockSpec((512, C), lambda e, j: (e * (CAP // 512) + j, 0)),
        out_shape=jax.ShapeDtypeStruct((NSLOT + 16384 + 8, C), jnp.float32),
    )(cnt, buf, eW1.astype(jnp.bfloat16), eb1r, eW2.astype(jnp.bfloat16), eb2r)

    g0, g1r_ = _make_sc_gather()(fbuf, i0r, i1r)

    out = pl.pallas_call(
        _combine_body,
        grid=(nt,),
        in_specs=[
            pl.BlockSpec((BT, C), lambda i: (i, 0)),
            pl.BlockSpec((BT, C), lambda i: (i, 0)),
            pl.BlockSpec((BT, C), lambda i: (i, 0)),
            pl.BlockSpec((BT, C), lambda i: (i, 0)),
            pl.BlockSpec((BT, 1), lambda i: (i, 0)),
            pl.BlockSpec((BT, 1), lambda i: (i, 0)),
            pl.BlockSpec((BT, E), lambda i: (i, 0)),
        ],
        out_specs=pl.BlockSpec((BT, C), lambda i: (i, 0)),
        out_shape=jax.ShapeDtypeStruct((N, C), jnp.float32),
    )(x1f, h, g0, g1r_, w0, w1, aux)

    return out.reshape(B, T, C)


# speedup vs baseline: 1.0951x; 1.0951x over previous
import jax
import jax.numpy as jnp
from jax import lax
from jax.experimental import pallas as pl
from jax.experimental.pallas import tpu as pltpu
from jax.experimental.pallas import tpu_sc as plsc

C = 128
E = 8
H = 8
D = 16
T = 32
NEG = -1e9
EPS = 1e-5
CAP = 4096
NSLOT = E * CAP
NW = 32
CH = 128


def _attn_body(x_ref, g_ref, b_ref, wq_ref, wk_ref, wv_ref, wp_ref, bp_ref, o_ref):
    xb = x_ref[...]
    BB = xb.shape[0]
    mu = jnp.mean(xb, axis=-1, keepdims=True)
    var = jnp.mean((xb - mu) ** 2, axis=-1, keepdims=True)
    xn = (xb - mu) / jnp.sqrt(var + EPS) * g_ref[...] + b_ref[...]
    x2 = xn.reshape(BB * T, C)
    q = jnp.dot(x2, wq_ref[...], preferred_element_type=jnp.float32).reshape(BB, T, H, D)
    k = jnp.dot(x2, wk_ref[...], preferred_element_type=jnp.float32).reshape(BB, T, H, D)
    v = jnp.dot(x2, wv_ref[...], preferred_element_type=jnp.float32).reshape(BB, T, H, D)
    row = jax.lax.broadcasted_iota(jnp.int32, (T, T), 0)
    col = jax.lax.broadcasted_iota(jnp.int32, (T, T), 1)
    causal = row >= col
    scale = C ** -0.5
    outs = []
    for h in range(H):
        qh = q[:, :, h, :]
        kh = k[:, :, h, :]
        vh = v[:, :, h, :]
        s = jax.lax.dot_general(qh, kh, (((2,), (2,)), ((0,), (0,))),
                                preferred_element_type=jnp.float32) * scale
        s = jnp.where(causal[None, :, :], s, NEG)
        m = jnp.max(s, axis=-1, keepdims=True)
        e = jnp.exp(s - m)
        p = e / jnp.sum(e, axis=-1, keepdims=True)
        oh = jax.lax.dot_general(p, vh, (((2,), (1,)), ((0,), (0,))),
                                 preferred_element_type=jnp.float32)
        outs.append(oh)
    o = jnp.concatenate(outs, axis=-1).reshape(BB * T, C)
    y = jnp.dot(o, wp_ref[...], preferred_element_type=jnp.float32) + bp_ref[...]
    o_ref[...] = xb + y.reshape(BB, T, C)


def _router_body(x1_ref, g_ref, b_ref, wcat_ref, bcat_ref, nz_ref, tri_ref,
                 h_ref, pos_ref, mask_ref, gate_ref, aux_ref, cnt_ref, acc_ref):
    i = pl.program_id(0)

    @pl.when(i == 0)
    def _():
        acc_ref[...] = jnp.zeros_like(acc_ref)

    xb = x1_ref[...]
    mu = jnp.mean(xb, axis=-1, keepdims=True)
    var = jnp.mean((xb - mu) ** 2, axis=-1, keepdims=True)
    hh = (xb - mu) / jnp.sqrt(var + EPS) * g_ref[...] + b_ref[...]
    h_ref[...] = hh

    lc = jnp.dot(hh, wcat_ref[...], preferred_element_type=jnp.float32) + bcat_ref[...]
    logits = lc[:, 0:E]
    nlog = lc[:, E:2 * E]
    sk = lc[:, 2 * E:2 * E + 1]
    softp = jnp.logaddexp(nlog, 0.0)
    noisy = logits + nz_ref[...] * softp

    r8 = jax.lax.broadcasted_iota(jnp.int32, (E, E), 0)
    c8 = jax.lax.broadcasted_iota(jnp.int32, (E, E), 1)
    l8 = (r8 <= c8).astype(jnp.float32)
    m1 = jnp.max(noisy, axis=-1, keepdims=True)
    eq1 = (noisy == m1).astype(jnp.float32)
    cs1 = jnp.dot(eq1, l8, preferred_element_type=jnp.float32)
    first = (eq1 > 0) & (cs1 == 1.0)
    noisy2 = jnp.where(first, -3e38, noisy)
    m2 = jnp.max(noisy2, axis=-1, keepdims=True)
    eq2 = (noisy2 == m2).astype(jnp.float32)
    cs2 = jnp.dot(eq2, l8, preferred_element_type=jnp.float32)
    second = (eq2 > 0) & (cs2 == 1.0)
    topk = first | second

    z = jnp.where(topk, jnp.exp(noisy - m1), 0.0)
    gate = z / jnp.sum(z, axis=-1, keepdims=True)
    gate_ref[...] = gate

    ns = (jax.nn.sigmoid(sk) <= 0.5).astype(jnp.float32)
    emask = topk.astype(jnp.float32) * ns
    mask_ref[...] = emask

    posb = jnp.dot(tri_ref[...], emask, preferred_element_type=jnp.float32)
    pos = posb + acc_ref[0:1, 0:E]
    pos_ref[...] = pos

    lane = jax.lax.broadcasted_iota(jnp.int32, emask.shape, 1)
    aux_ref[...] = jnp.where(lane == 0, ns, 0.0)

    new_cnt = acc_ref[0:1, 0:E] + jnp.sum(emask, axis=0, keepdims=True)
    new_ns = acc_ref[0:1, E:E + 1] + jnp.sum(ns, keepdims=True).reshape(1, 1)
    rowc = jnp.concatenate([new_cnt, new_ns, jnp.zeros((1, 16 - E - 1), jnp.float32)], axis=1)
    acc_ref[...] = rowc
    cnt_ref[...] = rowc


def _slots_body(cnt_ref, pos_ref, mask_ref, gate_ref,
                i0_ref, i1_ref, w0_ref, w1_ref):
    i = pl.program_id(0)
    tglob = (i * 512.0
             + jax.lax.broadcasted_iota(jnp.int32, (512, 1), 0).astype(jnp.float32))
    sent = float(NSLOT) + tglob
    ntok = cnt_ref[0, E]
    cap = jnp.floor(ntok * 2.0 / 8.0)
    maskf = mask_ref[...]
    pos = pos_ref[...]
    gate = gate_ref[...]
    r8 = jax.lax.broadcasted_iota(jnp.int32, (E, E), 0)
    c8 = jax.lax.broadcasted_iota(jnp.int32, (E, E), 1)
    l8 = (r8 <= c8).astype(jnp.float32)
    cs = jnp.dot(maskf, l8, preferred_element_type=jnp.float32)
    firstb = (maskf > 0) & (cs == 1.0)
    secondb = (maskf > 0) & (cs == 2.0)
    within = pos <= cap
    lanef = jax.lax.broadcasted_iota(jnp.int32, maskf.shape, 1).astype(jnp.float32)
    slotv = lanef * CAP + pos - 1.0

    sel0 = firstb & within
    sel0f = sel0.astype(jnp.float32)
    has0 = jnp.sum(sel0f, axis=-1, keepdims=True)
    d0 = jnp.sum(sel0f * slotv, axis=-1, keepdims=True)
    d0 = jnp.where(has0 > 0, d0, sent)
    w0 = jnp.sum(sel0f * gate, axis=-1, keepdims=True)

    sel1 = secondb & within
    sel1f = sel1.astype(jnp.float32)
    has1 = jnp.sum(sel1f, axis=-1, keepdims=True)
    d1 = jnp.sum(sel1f * slotv, axis=-1, keepdims=True)
    d1 = jnp.where(has1 > 0, d1, sent)
    w1 = jnp.sum(sel1f * gate, axis=-1, keepdims=True)

    i0_ref[...] = d0.astype(jnp.int32)
    i1_ref[...] = d1.astype(jnp.int32)
    w0_ref[...] = w0
    w1_ref[...] = w1


def _ffn_body(cnt_ref, buf_ref, w1_ref, b1_ref, w2_ref, b2_ref, o_ref):
    e = pl.program_id(0)
    j = pl.program_id(1)
    lane = jax.lax.broadcasted_iota(jnp.int32, (1, 16), 1)
    cnte = jnp.sum(jnp.where(lane == e, cnt_ref[...], 0.0))
    ntok = cnt_ref[0, E]
    cap = jnp.floor(ntok * 2.0 / 8.0)
    used = jnp.minimum(cnte, cap)

    @pl.when((j * 512.0) < used)
    def _():
        hh = buf_ref[...].astype(jnp.bfloat16)
        t = jnp.maximum(jnp.dot(hh, w1_ref[0], preferred_element_type=jnp.float32)
                        + b1_ref[0], 0.0)
        o_ref[...] = jnp.dot(t.astype(jnp.bfloat16), w2_ref[0],
                             preferred_element_type=jnp.float32) + b2_ref[0]


def _combine_body(x1_ref, h_ref, g0_ref, g1_ref, w0_ref, w1_ref, aux_ref, o_ref):
    w0 = w0_ref[...]
    w1 = w1_ref[...]
    upd = (jnp.where(w0 > 0, g0_ref[...], 0.0) * w0
           + jnp.where(w1 > 0, g1_ref[...], 0.0) * w1)
    ns = aux_ref[:, 0:1]
    o_ref[...] = x1_ref[...] + jnp.where(ns > 0.5, upd, h_ref[...])


def _make_sc_scatter():
    mesh = plsc.VectorSubcoreMesh(core_axis_name="c", subcore_axis_name="s")

    @pl.kernel(
        mesh=mesh,
        out_type=jax.ShapeDtypeStruct((NSLOT + 16384 + 8, C), jnp.float32),
        scratch_types=[
            pltpu.VMEM((512, C), jnp.float32),
            pltpu.VMEM((4, CH), jnp.int32),
            pltpu.VMEM((4, CH), jnp.int32),
            pltpu.SemaphoreType.DMA,
            pltpu.SemaphoreType.DMA,
        ],
    )
    def sc_scatter(h_hbm, i0_hbm, i1_hbm, buf_hbm, rows_v, idx0_v, idx1_v, semi, sem):
        wid = lax.axis_index("s") * 2 + lax.axis_index("c")
        base = wid * 512
        c0 = pltpu.async_copy(h_hbm.at[pl.ds(base, 512)], rows_v, semi)
        c1 = pltpu.async_copy(i0_hbm.at[wid], idx0_v, semi)
        c2 = pltpu.async_copy(i1_hbm.at[wid], idx1_v, semi)
        c0.wait()
        c1.wait()
        c2.wait()
        cps = []
        for j in range(4):
            src = rows_v.at[pl.ds(j * CH, CH)]
            cps.append(pltpu.async_copy(src, buf_hbm.at[idx0_v.at[j]], sem))
            cps.append(pltpu.async_copy(src, buf_hbm.at[idx1_v.at[j]], sem))
        for cp in cps:
            cp.wait()

    return sc_scatter


def _make_sc_gather():
    mesh = plsc.VectorSubcoreMesh(core_axis_name="c", subcore_axis_name="s")

    @pl.kernel(
        mesh=mesh,
        out_type=[
            jax.ShapeDtypeStruct((NW * 512, C), jnp.float32),
            jax.ShapeDtypeStruct((NW * 512, C), jnp.float32),
        ],
        scratch_types=[
            pltpu.VMEM((256, C), jnp.float32),
            pltpu.VMEM((256, C), jnp.float32),
            pltpu.VMEM((4, CH), jnp.int32),
            pltpu.VMEM((4, CH), jnp.int32),
            pltpu.SemaphoreType.DMA,
            pltpu.SemaphoreType.DMA,
        ],
    )
    def sc_gather(fbuf_hbm, i0_hbm, i1_hbm, g0_hbm, g1_hbm,
                  rows0_v, rows1_v, idx0_v, idx1_v, semi, sem):
        wid = lax.axis_index("s") * 2 + lax.axis_index("c")
        base = wid * 512
        c1 = pltpu.async_copy(i0_hbm.at[wid], idx0_v, semi)
        c2 = pltpu.async_copy(i1_hbm.at[wid], idx1_v, semi)
        c1.wait()
        c2.wait()
        for half in range(2):
            cps = []
            for j in range(2):
                jj = half * 2 + j
                cps.append(pltpu.async_copy(
                    fbuf_hbm.at[idx0_v.at[jj]], rows0_v.at[pl.ds(j * CH, CH)], sem))
                cps.append(pltpu.async_copy(
                    fbuf_hbm.at[idx1_v.at[jj]], rows1_v.at[pl.ds(j * CH, CH)], sem))
            for cp in cps:
                cp.wait()
            o0 = pltpu.async_copy(
                rows0_v, g0_hbm.at[pl.ds(base + half * 256, 256)], semi)
            o1 = pltpu.async_copy(
                rows1_v, g1_hbm.at[pl.ds(base + half * 256, 256)], semi)
            o0.wait()
            o1.wait()

    return sc_gather


def kernel(x, ln1_g, ln1_b, Wq, Wk, Wv, Wp, bp, ln2_g, ln2_b, We, be, Wn, bn,
           Ws, bs, eW1, eb1, eW2, eb2):
    B = x.shape[0]
    N = B * T
    BB = 64
    BT = 512
    nb = B // BB
    nt = N // BT

    wq = Wq.transpose(1, 0, 2).reshape(C, C)
    wk = Wk.transpose(1, 0, 2).reshape(C, C)
    wv = Wv.transpose(1, 0, 2).reshape(C, C)
    g1 = ln1_g.reshape(1, C)
    b1 = ln1_b.reshape(1, C)
    bpr = bp.reshape(1, C)

    x1 = pl.pallas_call(
        _attn_body,
        grid=(nb,),
        in_specs=[
            pl.BlockSpec((BB, T, C), lambda i: (i, 0, 0)),
            pl.BlockSpec((1, C), lambda i: (0, 0)),
            pl.BlockSpec((1, C), lambda i: (0, 0)),
            pl.BlockSpec((C, C), lambda i: (0, 0)),
            pl.BlockSpec((C, C), lambda i: (0, 0)),
            pl.BlockSpec((C, C), lambda i: (0, 0)),
            pl.BlockSpec((C, C), lambda i: (0, 0)),
            pl.BlockSpec((1, C), lambda i: (0, 0)),
        ],
        out_specs=pl.BlockSpec((BB, T, C), lambda i: (i, 0, 0)),
        out_shape=jax.ShapeDtypeStruct((B, T, C), jnp.float32),
    )(x, g1, b1, wq, wk, wv, Wp, bpr)

    x1f = x1.reshape(N, C)
    wcat = jnp.zeros((C, 32), jnp.float32)
    wcat = wcat.at[:, 0:E].set(We).at[:, E:2 * E].set(Wn).at[:, 2 * E:2 * E + 1].set(Ws)
    bcat = jnp.zeros((1, 32), jnp.float32)
    bcat = bcat.at[0, 0:E].set(be).at[0, E:2 * E].set(bn).at[0, 2 * E:2 * E + 1].set(bs)
    nz = jax.random.normal(jax.random.key(42), (B, T, E), dtype=jnp.float32).reshape(N, E)
    tri = jnp.tril(jnp.ones((BT, BT), jnp.float32))
    g2 = ln2_g.reshape(1, C)
    b2 = ln2_b.reshape(1, C)

    h, pos, maskf, gate, aux, cnt = pl.pallas_call(
        _router_body,
        grid=(nt,),
        in_specs=[
            pl.BlockSpec((BT, C), lambda i: (i, 0)),
            pl.BlockSpec((1, C), lambda i: (0, 0)),
            pl.BlockSpec((1, C), lambda i: (0, 0)),
            pl.BlockSpec((C, 32), lambda i: (0, 0)),
            pl.BlockSpec((1, 32), lambda i: (0, 0)),
            pl.BlockSpec((BT, E), lambda i: (i, 0)),
            pl.BlockSpec((BT, BT), lambda i: (0, 0)),
        ],
        out_specs=[
            pl.BlockSpec((BT, C), lambda i: (i, 0)),
            pl.BlockSpec((BT, E), lambda i: (i, 0)),
            pl.BlockSpec((BT, E), lambda i: (i, 0)),
            pl.BlockSpec((BT, E), lambda i: (i, 0)),
            pl.BlockSpec((BT, E), lambda i: (i, 0)),
            pl.BlockSpec((1, 16), lambda i: (0, 0)),
        ],
        out_shape=[
            jax.ShapeDtypeStruct((N, C), jnp.float32),
            jax.ShapeDtypeStruct((N, E), jnp.float32),
            jax.ShapeDtypeStruct((N, E), jnp.float32),
            jax.ShapeDtypeStruct((N, E), jnp.float32),
            jax.ShapeDtypeStruct((N, E), jnp.float32),
            jax.ShapeDtypeStruct((1, 16), jnp.float32),
        ],
        scratch_shapes=[pltpu.VMEM((1, 16), jnp.float32)],
    )(x1f, g2, b2, wcat, bcat, nz, tri)

    i0, i1, w0, w1 = pl.pallas_call(
        _slots_body,
        grid=(nt,),
        in_specs=[
            pl.BlockSpec((1, 16), lambda i: (0, 0)),
            pl.BlockSpec((BT, E), lambda i: (i, 0)),
            pl.BlockSpec((BT, E), lambda i: (i, 0)),
            pl.BlockSpec((BT, E), lambda i: (i, 0)),
        ],
        out_specs=[
            pl.BlockSpec((BT, 1), lambda i: (i, 0)),
            pl.BlockSpec((BT, 1), lambda i: (i, 0)),
            pl.BlockSpec((BT, 1), lambda i: (i, 0)),
            pl.BlockSpec((BT, 1), lambda i: (i, 0)),
        ],
        out_shape=[
            jax.ShapeDtypeStruct((N, 1), jnp.int32),
            jax.ShapeDtypeStruct((N, 1), jnp.int32),
            jax.ShapeDtypeStruct((N, 1), jnp.float32),
            jax.ShapeDtypeStruct((N, 1), jnp.float32),
        ],
    )(cnt, pos, maskf, gate)

    i0r = i0.reshape(NW, 4, CH)
    i1r = i1.reshape(NW, 4, CH)

    buf = _make_sc_scatter()(h, i0r, i1r)

    eb1r = eb1.reshape(E, 1, 4 * C)
    eb2r = eb2.reshape(E, 1, C)
    fbuf = pl.pallas_call(
        _ffn_body,
        grid=(E, CAP // 512),
        in_specs=[
            pl.BlockSpec((1, 16), lambda e, j: (0, 0)),
            pl.BlockSpec((512, C), lambda e, j: (e * (CAP // 512) + j, 0)),
            pl.BlockSpec((1, C, 4 * C), lambda e, j: (e, 0, 0)),
            pl.BlockSpec((1, 1, 4 * C), lambda e, j: (e, 0, 0)),
            pl.BlockSpec((1, 4 * C, C), lambda e, j: (e, 0, 0)),
            pl.BlockSpec((1, 1, C), lambda e, j: (e, 0, 0)),
        ],
        out_specs=pl.BlockSpec((512, C), lambda e, j: (e * (CAP // 512) + j, 0)),
        out_shape=jax.ShapeDtypeStruct((NSLOT + 16384 + 8, C), jnp.float32),
    )(cnt, buf, eW1.astype(jnp.bfloat16), eb1r, eW2.astype(jnp.bfloat16), eb2r)

    g0, g1r_ = _make_sc_gather()(fbuf, i0r, i1r)

    out = pl.pallas_call(
        _combine_body,
        grid=(nt,),
        in_specs=[
            pl.BlockSpec((BT, C), lambda i: (i, 0)),
            pl.BlockSpec((BT, C), lambda i: (i, 0)),
            pl.BlockSpec((BT, C), lambda i: (i, 0)),
            pl.BlockSpec((BT, C), lambda i: (i, 0)),
            pl.BlockSpec((BT, 1), lambda i: (i, 0)),
            pl.BlockSpec((BT, 1), lambda i: (i, 0)),
            pl.BlockSpec((BT, E), lambda i: (i, 0)),
        ],
        out_specs=pl.BlockSpec((BT, C), lambda i: (i, 0)),
        out_shape=jax.ShapeDtypeStruct((N, C), jnp.float32),
    )(x1f, h, g0, g1r_, w0, w1, aux)

    return out.reshape(B, T, C)
